# baseline (speedup 1.0000x reference)
import jax
import jax.numpy as jnp
from jax import lax
from jax.experimental import pallas as pl
from jax.experimental.pallas import tpu as pltpu
from jax.experimental.pallas import tpu_sc as plsc

N = 10000
E = 320000
D = 128
D2 = D // 2

NC = 2
NS = 16
NW = NC * NS

EPW = E // NW
K = 80
NCHUNK = EPW // K

KE = 128
EPS = 20480
EPAD = NS * EPS - E
NCHUNK_E = EPS // KE
NPAIR = NCHUNK_E // 2

_mesh = plsc.VectorSubcoreMesh(
    core_axis_name="c", subcore_axis_name="s", num_cores=NC, num_subcores=NS
)
_sc_params = pltpu.CompilerParams(needs_layout_passes=False)


def _sc_deg_body(dst3, ew3, zeros_hbm, degp_hbm, dsts_v, ew_v, sem, deg_sh):
    c = lax.axis_index("c")
    s = lax.axis_index("s")
    wid = s * NC + c

    @pl.when(s == 0)
    def _init():
        pltpu.sync_copy(zeros_hbm, deg_sh)

    pltpu.sync_copy(dst3.at[wid], dsts_v)
    pltpu.sync_copy(ew3.at[wid], ew_v)
    plsc.subcore_barrier()

    @pl.loop(0, NCHUNK)
    def _fire(ci):
        pltpu.async_copy(ew_v.at[ci], deg_sh.at[dsts_v.at[ci]], sem, add=True)

    @pl.loop(0, NCHUNK)
    def _drain(ci):
        pltpu.make_async_copy(ew_v.at[ci], deg_sh.at[dsts_v.at[ci]], sem).wait()

    plsc.subcore_barrier()

    @pl.when(s == 0)
    def _flush():
        pltpu.sync_copy(deg_sh, degp_hbm.at[c])


_sc_deg = pl.kernel(
    _sc_deg_body,
    out_type=jax.ShapeDtypeStruct((NC, N), jnp.float32),
    mesh=_mesh,
    scratch_types=[
        pltpu.VMEM((NCHUNK, K), jnp.int32),
        pltpu.VMEM((NCHUNK, K), jnp.float32),
        pltpu.SemaphoreType.DMA,
        pltpu.VMEM_SHARED((N,), jnp.float32),
    ],
    compiler_params=_sc_params,
)


NH = N // NC
NTRASH = 8
NROWS = NH + NTRASH


def _sc_edge_body(gh_hbm, src3, dst4, ew3, zeros_hbm, accp_hbm,
                  srcs_v, dsts_v, ew0, ew1, rows0, rows1, srows,
                  gsem0, gsem1, esem0, esem1, acc_sh):
    c = lax.axis_index("c")
    s = lax.axis_index("s")

    @pl.when(s == 0)
    def _init():
        pltpu.sync_copy(zeros_hbm, acc_sh)

    pltpu.sync_copy(src3.at[s], srcs_v)
    pltpu.sync_copy(dst4.at[c].at[s], dsts_v)
    plsc.subcore_barrier()

    def _fire_gather(ci, rows, gsem):
        pltpu.async_copy(gh_hbm.at[srcs_v.at[ci]], rows, gsem)

    def _wait_gather(ci, rows, gsem):
        pltpu.make_async_copy(gh_hbm.at[srcs_v.at[ci]], rows, gsem).wait()

    def _fire_ew(ci, ew_b, esem):
        pltpu.async_copy(ew3.at[s].at[ci], ew_b, esem)

    def _wait_ew(ci, ew_b, esem):
        pltpu.make_async_copy(ew3.at[s].at[ci], ew_b, esem).wait()

    def _scale(rows, ew_b):
        @pl.loop(0, KE)
        def _edge(e):
            w16 = plsc.load_gather(ew_b, [jnp.full((16,), e, jnp.int32)])
            for p in range(D // 16):
                srows[e, pl.ds(p * 16, 16)] = (
                    rows[e, pl.ds(p * 16, 16)] * w16)

    def _half(pi, c_i, rows, ew_b, gsem, esem):
        _wait_gather(c_i, rows, gsem)
        _wait_ew(c_i, ew_b, esem)
        _scale(rows, ew_b)
        pltpu.sync_copy(srows, acc_sh.at[dsts_v.at[c_i]], add=True)

        @pl.when(pi < NPAIR - 1)
        def _():
            _fire_gather(c_i + 2, rows, gsem)
            _fire_ew(c_i + 2, ew_b, esem)

    _fire_ew(0, ew0, esem0)
    _fire_ew(1, ew1, esem1)
    _fire_gather(0, rows0, gsem0)
    _fire_gather(1, rows1, gsem1)

    @pl.loop(0, NPAIR)
    def _pair(pi):
        c0 = 2 * pi
        _half(pi, c0, rows0, ew0, gsem0, esem0)
        _half(pi, c0 + 1, rows1, ew1, gsem1, esem1)

    plsc.subcore_barrier()

    @pl.when(s == 0)
    def _flush():
        pltpu.sync_copy(acc_sh.at[pl.ds(0, NH)],
                        accp_hbm.at[pl.ds(c * NH, NH)])


_sc_edge = pl.kernel(
    _sc_edge_body,
    out_type=jax.ShapeDtypeStruct((N, D), jnp.float32),
    mesh=_mesh,
    scratch_types=[
        pltpu.VMEM((NCHUNK_E, KE), jnp.int32),
        pltpu.VMEM((NCHUNK_E, KE), jnp.int32),
        pltpu.VMEM((KE,), jnp.float32),
        pltpu.VMEM((KE,), jnp.float32),
        pltpu.VMEM((KE, D), jnp.float32),
        pltpu.VMEM((KE, D), jnp.float32),
        pltpu.VMEM((KE, D), jnp.float32),
        pltpu.SemaphoreType.DMA,
        pltpu.SemaphoreType.DMA,
        pltpu.SemaphoreType.DMA,
        pltpu.SemaphoreType.DMA,
        pltpu.VMEM_SHARED((NROWS, D), jnp.float32),
    ],
    compiler_params=_sc_params,
)

BR = 512


def _tc_lin_body(x_ref, w_ref, degp_ref, g_ref):
    deg = degp_ref[0, :] + degp_ref[1, :] + 2.0
    dis = lax.rsqrt(deg)
    h = jnp.dot(x_ref[...], w_ref[...], preferred_element_type=jnp.float32)
    g_ref[...] = h * dis[:, None]


def _tc_lin(x, w, degp):
    return pl.pallas_call(
        _tc_lin_body,
        grid=(pl.cdiv(N, BR),),
        in_specs=[
            pl.BlockSpec((BR, D), lambda i: (i, 0)),
            pl.BlockSpec((D, D), lambda i: (0, 0)),
            pl.BlockSpec((2, BR), lambda i: (0, i)),
        ],
        out_specs=pl.BlockSpec((BR, D), lambda i: (i, 0)),
        out_shape=jax.ShapeDtypeStruct((N, D), jnp.float32),
    )(x, w, degp)


def _tc_comb_body(acc_ref, g_ref, degp_ref, b_ref, o_ref):
    deg = degp_ref[0, :] + degp_ref[1, :] + 2.0
    dis = lax.rsqrt(deg)
    o_ref[...] = (acc_ref[...] + 2.0 * g_ref[...]) * dis[:, None] + b_ref[...]


def _tc_comb(acc, g, degp, b2):
    return pl.pallas_call(
        _tc_comb_body,
        grid=(pl.cdiv(N, BR),),
        in_specs=[
            pl.BlockSpec((BR, D), lambda i: (i, 0)),
            pl.BlockSpec((BR, D), lambda i: (i, 0)),
            pl.BlockSpec((2, BR), lambda i: (0, i)),
            pl.BlockSpec((1, D), lambda i: (0, 0)),
        ],
        out_specs=pl.BlockSpec((BR, D), lambda i: (i, 0)),
        out_shape=jax.ShapeDtypeStruct((N, D), jnp.float32),
    )(acc, g, degp, b2)


@jax.jit
def kernel(x, edge_index, edge_weight, W, b):
    src_i = edge_index[0].astype(jnp.int32)
    dst_i = edge_index[1].astype(jnp.int32)
    ew_f = edge_weight.astype(jnp.float32)
    zeros1 = jnp.zeros((N,), jnp.float32)
    zeros2 = jnp.zeros((NROWS, D), jnp.float32)
    degp = _sc_deg(dst_i.reshape(NW, NCHUNK, K), ew_f.reshape(NW, NCHUNK, K),
                   zeros1)
    g = _tc_lin(x, W, degp)
    pad_i = jnp.zeros((EPAD,), jnp.int32)
    src_pad = jnp.concatenate([src_i, pad_i])
    dst_pad = jnp.concatenate([dst_i, pad_i])
    trash = NH + (dst_pad & (NTRASH - 1))
    dst_c0 = jnp.where(dst_pad < NH, dst_pad, trash)
    dst_c1 = jnp.where(dst_pad >= NH, dst_pad - NH, trash)
    dst4 = jnp.stack([dst_c0, dst_c1]).reshape(NC, NS, NCHUNK_E, KE)
    src_p = src_pad.reshape(NS, NCHUNK_E, KE)
    ew_p = jnp.concatenate(
        [ew_f, jnp.zeros((EPAD,), jnp.float32)]).reshape(NS, NCHUNK_E, KE)
    acc = _sc_edge(g, src_p, dst4, ew_p, zeros2)
    return _tc_comb(acc, g, degp, b.reshape(1, D))


# speedup vs baseline: 1.9063x; 1.9063x over previous
import jax
import jax.numpy as jnp
from jax import lax
from jax.experimental import pallas as pl
from jax.experimental.pallas import tpu as pltpu
from jax.experimental.pallas import tpu_sc as plsc

N = 10000
E = 320000
D = 128

NC = 2
NS = 16
NW = NC * NS
EPW = E // NW
K = 80
NCHUNK = EPW // K
GRP = K // 16

_mesh = plsc.VectorSubcoreMesh(
    core_axis_name="c", subcore_axis_name="s", num_cores=NC, num_subcores=NS
)
_sc_params = pltpu.CompilerParams(needs_layout_passes=False)


def _sc_deg_body(dst3, ew3, zeros_hbm, degp_hbm, dsts_v, ew_v, sem, deg_sh):
    c = lax.axis_index("c")
    s = lax.axis_index("s")
    wid = s * NC + c

    @pl.when(s == 0)
    def _init():
        pltpu.sync_copy(zeros_hbm, deg_sh)

    pltpu.sync_copy(dst3.at[wid], dsts_v)
    pltpu.sync_copy(ew3.at[wid], ew_v)
    plsc.subcore_barrier()

    @pl.loop(0, NCHUNK)
    def _fire(ci):
        pltpu.async_copy(ew_v.at[ci], deg_sh.at[dsts_v.at[ci]], sem, add=True)

    @pl.loop(0, NCHUNK)
    def _drain(ci):
        pltpu.make_async_copy(ew_v.at[ci], deg_sh.at[dsts_v.at[ci]], sem).wait()

    plsc.subcore_barrier()

    @pl.when(s == 0)
    def _flush():
        pltpu.sync_copy(deg_sh, degp_hbm.at[c])


_sc_deg = pl.kernel(
    _sc_deg_body,
    out_type=jax.ShapeDtypeStruct((NC, N), jnp.float32),
    mesh=_mesh,
    scratch_types=[
        pltpu.VMEM((NCHUNK, K), jnp.int32),
        pltpu.VMEM((NCHUNK, K), jnp.float32),
        pltpu.SemaphoreType.DMA,
        pltpu.VMEM_SHARED((N,), jnp.float32),
    ],
    compiler_params=_sc_params,
)


KE = 100
NCHUNK_E = EPW // KE
NPAIR = NCHUNK_E // 2


def _sc_edge_body(g_hbm, src3, dst3, ew3, zeros_hbm, accp_hbm,
                  src0, src1, dst0, dst1, ew0, ew1,
                  rows0, rows1, srows_v, gsem0, gsem1, acc_sh):
    c = lax.axis_index("c")
    s = lax.axis_index("s")
    wid = s * NC + c

    @pl.when(s == 0)
    def _init():
        pltpu.sync_copy(zeros_hbm, acc_sh)

    plsc.subcore_barrier()

    def _load_idx(ci, src_b, dst_b, ew_b):
        pltpu.sync_copy(src3.at[wid].at[ci], src_b)
        pltpu.sync_copy(dst3.at[wid].at[ci], dst_b)
        pltpu.sync_copy(ew3.at[wid].at[ci], ew_b)

    def _fire_gather(src_b, rows, gsem):
        pltpu.async_copy(g_hbm.at[src_b], rows, gsem)

    def _wait_gather(src_b, rows, gsem):
        pltpu.make_async_copy(g_hbm.at[src_b], rows, gsem).wait()

    def _half(pi, c_i, src_b, dst_b, ew_b, rows, gsem):
        _wait_gather(src_b, rows, gsem)

        @pl.loop(0, KE)
        def _edge(e):
            w16 = plsc.load_gather(ew_b, [jnp.full((16,), e, jnp.int32)])
            for p in range(D // 16):
                srows_v[e, pl.ds(p * 16, 16)] = (
                    rows[e, pl.ds(p * 16, 16)] * w16)

        pltpu.sync_copy(srows_v, acc_sh.at[dst_b], add=True)

        @pl.when(pi < NPAIR - 1)
        def _():
            _load_idx(c_i + 2, src_b, dst_b, ew_b)
            _fire_gather(src_b, rows, gsem)

    _load_idx(0, src0, dst0, ew0)
    _fire_gather(src0, rows0, gsem0)
    _load_idx(1, src1, dst1, ew1)
    _fire_gather(src1, rows1, gsem1)

    @pl.loop(0, NPAIR)
    def _pair(pi):
        c0 = 2 * pi
        _half(pi, c0, src0, dst0, ew0, rows0, gsem0)
        _half(pi, c0 + 1, src1, dst1, ew1, rows1, gsem1)

    plsc.subcore_barrier()

    @pl.when(s == 0)
    def _flush():
        pltpu.sync_copy(acc_sh, accp_hbm.at[c])


_sc_edge = pl.kernel(
    _sc_edge_body,
    out_type=jax.ShapeDtypeStruct((NC, N, D), jnp.float32),
    mesh=_mesh,
    scratch_types=[
        pltpu.VMEM((KE,), jnp.int32),
        pltpu.VMEM((KE,), jnp.int32),
        pltpu.VMEM((KE,), jnp.int32),
        pltpu.VMEM((KE,), jnp.int32),
        pltpu.VMEM((KE,), jnp.float32),
        pltpu.VMEM((KE,), jnp.float32),
        pltpu.VMEM((KE, D), jnp.float32),
        pltpu.VMEM((KE, D), jnp.float32),
        pltpu.VMEM((KE, D), jnp.float32),
        pltpu.SemaphoreType.DMA,
        pltpu.SemaphoreType.DMA,
        pltpu.VMEM_SHARED((N, D), jnp.float32),
    ],
    compiler_params=_sc_params,
)

BR = 512


def _tc_lin_body(x_ref, w_ref, degp_ref, g_ref):
    deg = degp_ref[0, :] + degp_ref[1, :] + 2.0
    dis = lax.rsqrt(deg)
    h = jnp.dot(x_ref[...], w_ref[...], preferred_element_type=jnp.float32)
    g_ref[...] = h * dis[:, None]


def _tc_lin(x, w, degp):
    return pl.pallas_call(
        _tc_lin_body,
        grid=(pl.cdiv(N, BR),),
        in_specs=[
            pl.BlockSpec((BR, D), lambda i: (i, 0)),
            pl.BlockSpec((D, D), lambda i: (0, 0)),
            pl.BlockSpec((2, BR), lambda i: (0, i)),
        ],
        out_specs=pl.BlockSpec((BR, D), lambda i: (i, 0)),
        out_shape=jax.ShapeDtypeStruct((N, D), jnp.float32),
    )(x, w, degp)


def _tc_comb_body(accp_ref, g_ref, degp_ref, b_ref, o_ref):
    deg = degp_ref[0, :] + degp_ref[1, :] + 2.0
    dis = lax.rsqrt(deg)
    t = accp_ref[0] + accp_ref[1] + 2.0 * g_ref[...]
    o_ref[...] = t * dis[:, None] + b_ref[...]


def _tc_comb(accp, g, degp, b2):
    return pl.pallas_call(
        _tc_comb_body,
        grid=(pl.cdiv(N, BR),),
        in_specs=[
            pl.BlockSpec((2, BR, D), lambda i: (0, i, 0)),
            pl.BlockSpec((BR, D), lambda i: (i, 0)),
            pl.BlockSpec((2, BR), lambda i: (0, i)),
            pl.BlockSpec((1, D), lambda i: (0, 0)),
        ],
        out_specs=pl.BlockSpec((BR, D), lambda i: (i, 0)),
        out_shape=jax.ShapeDtypeStruct((N, D), jnp.float32),
    )(accp, g, degp, b2)


@jax.jit
def kernel(x, edge_index, edge_weight, W, b):
    src_i = edge_index[0].astype(jnp.int32)
    dst_i = edge_index[1].astype(jnp.int32)
    ew_f = edge_weight.astype(jnp.float32)
    zeros1 = jnp.zeros((N,), jnp.float32)
    zeros2 = jnp.zeros((N, D), jnp.float32)
    degp = _sc_deg(dst_i.reshape(NW, NCHUNK, K), ew_f.reshape(NW, NCHUNK, K),
                   zeros1)
    g = _tc_lin(x, W, degp)
    accp = _sc_edge(g, src_i.reshape(NW, NCHUNK_E, KE),
                    dst_i.reshape(NW, NCHUNK_E, KE),
                    ew_f.reshape(NW, NCHUNK_E, KE), zeros2)
    return _tc_comb(accp, g, degp, b.reshape(1, D))
